# baseline (speedup 1.0000x reference)
import functools
import math

import jax
import jax.numpy as jnp
from jax import lax
from jax.experimental import pallas as pl
from jax.experimental.pallas import tpu as pltpu
from jax.experimental.pallas import tpu_sc as plsc

_D = 64
_SCALE = 8.0
_LANES = 16
_NC = 2
_NS = 16
_NW = _NC * _NS
_C = 128
_NBUF = 5
_LOOK = 3


@jax.jit
def _gather_scale(idx1d, lut3):
    B = 4096
    S = idx1d.shape[0] // B
    nchunk = S
    ngroup = nchunk // _NBUF
    mesh = plsc.VectorSubcoreMesh(core_axis_name="c", subcore_axis_name="s")

    @functools.partial(
        pl.kernel,
        out_type=jax.ShapeDtypeStruct((S * B, _D), jnp.float32),
        mesh=mesh,
        scratch_types=[
            pltpu.VMEM((_NBUF, _C), jnp.int32),
            pltpu.VMEM((_NBUF, _C, _D), jnp.float32),
            pltpu.SemaphoreType.DMA((_NBUF,)),
            pltpu.SemaphoreType.DMA((_NBUF,)),
            pltpu.SemaphoreType.DMA((_NBUF,)),
        ],
        compiler_params=pltpu.CompilerParams(use_tc_tiling_on_sc=True),
    )
    def body(idx_hbm, lut_hbm, out_hbm, idx_v, bufs, isem, gsem, ssem):
        wid = lax.axis_index("s") * _NC + lax.axis_index("c")
        col0 = wid * _C

        def fire_idx(g, b):
            pltpu.async_copy(
                idx_hbm.at[pl.ds(g * B + col0, _C)], idx_v.at[b], isem.at[b]
            )

        def wait_idx(b):
            pltpu.make_async_copy(
                idx_hbm.at[pl.ds(0, _C)], idx_v.at[b], isem.at[b]
            ).wait()

        def fire_gather(b):
            for t in range(_C // _LANES):
                vec = idx_v[b, pl.ds(t * _LANES, _LANES)]
                for i2 in range(_LANES):
                    r = vec[i2]
                    pltpu.async_copy(
                        lut_hbm.at[r >> 3, r & 7],
                        bufs.at[b, t * _LANES + i2],
                        gsem.at[b],
                    )

        def wait_gather(b):
            pltpu.make_async_copy(
                out_hbm.at[pl.ds(0, _C)], bufs.at[b], gsem.at[b]
            ).wait()

        def fire_scatter(g, b):
            pltpu.async_copy(
                bufs.at[b],
                out_hbm.at[pl.ds((g * (B // _C) + wid) * _C, _C)],
                ssem.at[b],
            )

        def wait_scatter(b):
            pltpu.make_async_copy(
                bufs.at[b], out_hbm.at[pl.ds(0, _C)], ssem.at[b]
            ).wait()

        for b in range(_LOOK):
            fire_idx(b, b)
        for b in range(2):
            wait_idx(b)
            fire_gather(b)

        def group(go, carry):
            for b in range(_NBUF):
                g = go * _NBUF + b
                p = g + _LOOK
                pbi = (b + _LOOK) % _NBUF
                c = g + 2
                bc = (b + 2) % _NBUF

                @pl.when(p < nchunk)
                def _():
                    fire_idx(p, pbi)

                @pl.when(jnp.logical_and(c >= _NBUF, c < nchunk))
                def _():
                    wait_scatter(bc)

                @pl.when(c < nchunk)
                def _():
                    wait_idx(bc)
                    fire_gather(bc)

                wait_gather(b)

                def row(i, c2):
                    for i2 in range(2):
                        for j in range(_D // _LANES):
                            sl = bufs[b, 2 * i + i2, pl.ds(j * _LANES, _LANES)]
                            bufs[b, 2 * i + i2, pl.ds(j * _LANES, _LANES)] = (
                                sl * _SCALE
                            )
                    return c2

                lax.fori_loop(0, _C // 2, row, 0)
                fire_scatter(g, b)
            return carry

        lax.fori_loop(0, ngroup, group, 0)

        for b in range(_NBUF):
            wait_scatter(b)

    return body(idx1d, lut3)


def kernel(x, lut):
    r, s = x.shape
    idx1d = x.T.reshape(r * s).astype(jnp.int32)
    lut3 = lut.reshape(lut.shape[0] // 8, 8, _D)
    out2 = _gather_scale(idx1d, lut3)
    return jnp.transpose(out2.reshape(s, r, _D), (1, 0, 2))


# speedup vs baseline: 1.0183x; 1.0183x over previous
import functools
import math

import jax
import jax.numpy as jnp
from jax import lax
from jax.experimental import pallas as pl
from jax.experimental.pallas import tpu as pltpu
from jax.experimental.pallas import tpu_sc as plsc

_D = 64
_SCALE = 8.0
_LANES = 16
_NC = 2
_NS = 16
_NW = _NC * _NS
_C = 128
_NBUF = 5
_LOOK = 3


@jax.jit
def _gather_scale(idx1d, lut3):
    B = 4096
    S = idx1d.shape[0] // B
    nchunk = S
    ngroup = nchunk // _NBUF
    mesh = plsc.VectorSubcoreMesh(core_axis_name="c", subcore_axis_name="s")

    @functools.partial(
        pl.kernel,
        out_type=jax.ShapeDtypeStruct((S * B, _D), jnp.float32),
        mesh=mesh,
        scratch_types=[
            pltpu.VMEM((_NBUF, _C), jnp.int32),
            pltpu.VMEM((_NBUF, _C, _D), jnp.float32),
            pltpu.SemaphoreType.DMA((_NBUF,)),
            pltpu.SemaphoreType.DMA((_NBUF,)),
            pltpu.SemaphoreType.DMA((_NBUF,)),
        ],
        compiler_params=pltpu.CompilerParams(use_tc_tiling_on_sc=True),
    )
    def body(idx_hbm, lut_hbm, out_hbm, idx_v, bufs, isem, gsem, ssem):
        wid = lax.axis_index("s") * _NC + lax.axis_index("c")
        col0 = wid * _C

        def fire_idx(g, b):
            pltpu.async_copy(
                idx_hbm.at[pl.ds(g * B + col0, _C)], idx_v.at[b], isem.at[b]
            )

        def wait_idx(b):
            pltpu.make_async_copy(
                idx_hbm.at[pl.ds(0, _C)], idx_v.at[b], isem.at[b]
            ).wait()

        def fire_gather(b):
            def grp(t, carry):
                vec = idx_v[b, pl.ds(t * _LANES, _LANES)]
                for i2 in range(_LANES):
                    r = vec[i2]
                    pltpu.async_copy(
                        lut_hbm.at[r >> 3, r & 7],
                        bufs.at[b, t * _LANES + i2],
                        gsem.at[b],
                    )
                return carry

            lax.fori_loop(0, _C // _LANES, grp, 0)

        def wait_gather(b):
            pltpu.make_async_copy(
                out_hbm.at[pl.ds(0, _C)], bufs.at[b], gsem.at[b]
            ).wait()

        def fire_scatter(g, b):
            pltpu.async_copy(
                bufs.at[b],
                out_hbm.at[pl.ds((g * (B // _C) + wid) * _C, _C)],
                ssem.at[b],
            )

        def wait_scatter(b):
            pltpu.make_async_copy(
                bufs.at[b], out_hbm.at[pl.ds(0, _C)], ssem.at[b]
            ).wait()

        for b in range(_LOOK):
            fire_idx(b, b)
        wait_idx(0)
        fire_gather(0)

        def group(go, carry):
            for b in range(_NBUF):
                g = go * _NBUF + b
                p = g + _LOOK
                pbi = (b + _LOOK) % _NBUF
                c = g + 1
                bc = (b + 1) % _NBUF

                @pl.when(p < nchunk)
                def _():
                    fire_idx(p, pbi)

                @pl.when(jnp.logical_and(c >= _NBUF, c < nchunk))
                def _():
                    wait_scatter(bc)

                @pl.when(c < nchunk)
                def _():
                    wait_idx(bc)
                    fire_gather(bc)

                wait_gather(b)

                def row(i, c2):
                    for j in range(_D // _LANES):
                        sl = bufs[b, i, pl.ds(j * _LANES, _LANES)]
                        bufs[b, i, pl.ds(j * _LANES, _LANES)] = sl * _SCALE
                    return c2

                lax.fori_loop(0, _C, row, 0)
                fire_scatter(g, b)
            return carry

        lax.fori_loop(0, ngroup, group, 0)

        for b in range(_NBUF):
            wait_scatter(b)

    return body(idx1d, lut3)


def kernel(x, lut):
    r, s = x.shape
    idx1d = x.T.reshape(r * s).astype(jnp.int32)
    lut3 = lut.reshape(lut.shape[0] // 8, 8, _D)
    out2 = _gather_scale(idx1d, lut3)
    return jnp.transpose(out2.reshape(s, r, _D), (1, 0, 2))
